# Initial kernel scaffold; baseline (speedup 1.0000x reference)
import jax
import jax.numpy as jnp
from jax.experimental import pallas as pl


def kernel(feats16_0, feats16_1, feats8_0, feats8_1, feats4_0, feats4_1, t64, t32, t16):
    raise NotImplementedError("write your pallas kernel here")


# speedup vs baseline: 3.8082x; 3.8082x over previous
import functools

import numpy as np
import jax
import jax.numpy as jnp
from jax.experimental import pallas as pl
from jax.experimental.pallas import tpu as pltpu

L = 400
C = 256
UP = 1600
TILE = 160
LT = 100
LTP = 128


def _pool1d_matrix(l_in, l_out):
    w = np.zeros((l_in, l_out), dtype=np.float32)
    for i in range(l_out):
        s = (i * l_in) // l_out
        e = -(((-(i + 1)) * l_in) // l_out)
        w[s:e, i] = 1.0 / (e - s)
    return w


def _bilinear_matrix(l_in, l_out):
    ys = np.linspace(0.0, l_in - 1.0, l_out, dtype=np.float32)
    y0 = np.floor(ys).astype(np.int64)
    y1 = np.minimum(y0 + 1, l_in - 1)
    wy = ys - y0.astype(np.float32)
    a = np.zeros((l_out, l_in), dtype=np.float32)
    for i in range(l_out):
        a[i, y0[i]] += 1.0 - wy[i]
        a[i, y1[i]] += wy[i]
    return a


def _spatial_pool_matrix():
    p = np.zeros((LTP, L), dtype=np.float32)
    for j in range(LT):
        rr, cc = divmod(j, 10)
        for dr in range(2):
            for dc in range(2):
                p[j, (2 * rr + dr) * 20 + (2 * cc + dc)] = 0.25
    return p


@functools.cache
def _consts():
    a = _bilinear_matrix(L, UP)
    wa = _pool1d_matrix(C + 1, C)
    wb = _pool1d_matrix(2 * C, C)
    wal = np.zeros((8, C), dtype=np.float32)
    wal[0] = wa[C]
    return (a, a.T.copy(), wa[:C].copy(), wal, wb[:C].copy(), wb[C:].copy(),
            _spatial_pool_matrix())


def _dot(x, y):
    return jax.lax.dot_general(x, y, (((1,), (0,)), ((), ())),
                               precision=jax.lax.Precision.HIGHEST,
                               preferred_element_type=jnp.float32)


def _fused_kernel(x0_ref, x1_ref, t32_ref, a_ref, b_ref, p_ref, wat_ref,
                  wal_ref, wbt_ref, wbb_ref,
                  conf_f_ref, f0_ref, f1_ref, t32p_ref, tmp_ref):
    g = pl.program_id(0)

    @pl.when(g == 0)
    def _phase0():
        x0 = x0_ref[...]
        x1 = x1_ref[...]
        scale = 1.0 / (C ** 0.5)
        conf = jax.lax.dot_general(x0, x1, (((1,), (1,)), ((), ())),
                                   preferred_element_type=jnp.float32) * scale
        confT = jax.lax.dot_general(x1, x0, (((1,), (1,)), ((), ())),
                                    preferred_element_type=jnp.float32) * scale

        def dual_softmax(cm):
            e2 = jnp.exp(cm - jnp.max(cm, axis=1, keepdims=True))
            sm2 = e2 / jnp.sum(e2, axis=1, keepdims=True)
            e1 = jnp.exp(cm - jnp.max(cm, axis=0, keepdims=True))
            sm1 = e1 / jnp.sum(e1, axis=0, keepdims=True)
            return sm1 * sm2

        iota_s = jax.lax.broadcasted_iota(jnp.int32, (L, L), 1)

        def row_argmax(cm):
            m = jnp.max(cm, axis=1, keepdims=True)
            return jnp.min(jnp.where(cm == m, iota_s, jnp.int32(1 << 30)),
                           axis=1, keepdims=True)

        confm = dual_softmax(conf)
        confmT = dual_softmax(confT)
        idx = row_argmax(confm)
        idy = row_argmax(confmT)

        onehot = (iota_s == idx).astype(jnp.float32)
        gsel = _dot(onehot, x1)
        ft100 = _dot(p_ref[...], (x0 + gsel) * 0.5)

        t32p_ref[...] = (_dot(t32_ref[...], wbt_ref[...]) +
                         _dot(ft100, wbb_ref[...]))
        wal = wal_ref[0:1, :]
        idxf = idx.astype(jnp.float32).astype(jnp.bfloat16).astype(jnp.float32)
        idyf = idy.astype(jnp.float32).astype(jnp.bfloat16).astype(jnp.float32)
        f0_ref[...] = _dot(x0, wat_ref[...]) + idxf * wal
        f1_ref[...] = _dot(x1, wat_ref[...]) + idyf * wal
        tmp_ref[...] = _dot(conf, b_ref[...])

    conf_f_ref[...] = _dot(a_ref[...], tmp_ref[...])


def kernel(feats16_0, feats16_1, feats8_0, feats8_1, feats4_0, feats4_1,
           t64, t32, t16):
    del feats8_0, feats8_1, feats4_0, feats4_1, t64, t16
    a, b, wat, wal, wbt, wbb, p = (jnp.asarray(c) for c in _consts())
    x0 = feats16_0[0]
    x1 = feats16_1[0]
    t32pad = jnp.pad(t32[0], ((0, LTP - LT), (0, 0)))

    grid = (UP // TILE,)
    conf_f, f0, f1, t32p = pl.pallas_call(
        _fused_kernel,
        grid=grid,
        in_specs=[
            pl.BlockSpec((L, C), lambda g: (0, 0)),
            pl.BlockSpec((L, C), lambda g: (0, 0)),
            pl.BlockSpec((LTP, C), lambda g: (0, 0)),
            pl.BlockSpec((TILE, L), lambda g: (g, 0)),
            pl.BlockSpec((L, UP), lambda g: (0, 0)),
            pl.BlockSpec((LTP, L), lambda g: (0, 0)),
            pl.BlockSpec((C, C), lambda g: (0, 0)),
            pl.BlockSpec((8, C), lambda g: (0, 0)),
            pl.BlockSpec((C, C), lambda g: (0, 0)),
            pl.BlockSpec((C, C), lambda g: (0, 0)),
        ],
        out_specs=[
            pl.BlockSpec((TILE, UP), lambda g: (g, 0)),
            pl.BlockSpec((L, C), lambda g: (0, 0)),
            pl.BlockSpec((L, C), lambda g: (0, 0)),
            pl.BlockSpec((LTP, C), lambda g: (0, 0)),
        ],
        out_shape=[
            jax.ShapeDtypeStruct((UP, UP), jnp.float32),
            jax.ShapeDtypeStruct((L, C), jnp.float32),
            jax.ShapeDtypeStruct((L, C), jnp.float32),
            jax.ShapeDtypeStruct((LTP, C), jnp.float32),
        ],
        scratch_shapes=[pltpu.VMEM((L, UP), jnp.float32)],
    )(x0, x1, t32pad, a, b, p, wat, wal, wbt, wbb)

    return (conf_f[None, None], f0[None], f1[None],
            t32p[:LT, None, :])


# speedup vs baseline: 6.8580x; 1.8009x over previous
import functools

import numpy as np
import jax
import jax.numpy as jnp
from jax.experimental import pallas as pl
from jax.experimental.pallas import tpu as pltpu

L = 400
C = 256
UP = 1600
TILE = 160
LT = 100
LTP = 128


def _pool1d_matrix(l_in, l_out):
    w = np.zeros((l_in, l_out), dtype=np.float32)
    for i in range(l_out):
        s = (i * l_in) // l_out
        e = -(((-(i + 1)) * l_in) // l_out)
        w[s:e, i] = 1.0 / (e - s)
    return w


def _bilinear_matrix(l_in, l_out):
    ys = np.linspace(0.0, l_in - 1.0, l_out, dtype=np.float32)
    y0 = np.floor(ys).astype(np.int64)
    y1 = np.minimum(y0 + 1, l_in - 1)
    wy = ys - y0.astype(np.float32)
    a = np.zeros((l_out, l_in), dtype=np.float32)
    for i in range(l_out):
        a[i, y0[i]] += 1.0 - wy[i]
        a[i, y1[i]] += wy[i]
    return a


def _spatial_pool_matrix():
    p = np.zeros((LTP, L), dtype=np.float32)
    for j in range(LT):
        rr, cc = divmod(j, 10)
        for dr in range(2):
            for dc in range(2):
                p[j, (2 * rr + dr) * 20 + (2 * cc + dc)] = 0.25
    return p


@functools.cache
def _consts():
    a = _bilinear_matrix(L, UP)
    wa = _pool1d_matrix(C + 1, C)
    wb = _pool1d_matrix(2 * C, C)
    wal = np.zeros((8, C), dtype=np.float32)
    wal[0] = wa[C]
    return (a, a.T.copy(), wa[:C].copy(), wal, wb[:C].copy(), wb[C:].copy(),
            _spatial_pool_matrix())


def _dot(x, y):
    return jax.lax.dot_general(x, y, (((1,), (0,)), ((), ())),
                               precision=jax.lax.Precision.HIGHEST,
                               preferred_element_type=jnp.float32)


def _fused_kernel(x0_ref, x1_ref, t32_ref, a_ref, b_ref, p_ref, wat_ref,
                  wal_ref, wbt_ref, wbb_ref,
                  conf_f_ref, f0_ref, f1_ref, t32p_ref, tmp_ref):
    g = pl.program_id(0)

    @pl.when(g == 0)
    def _phase0():
        x0 = x0_ref[...]
        x1 = x1_ref[...]
        scale = 1.0 / (C ** 0.5)
        conf = jax.lax.dot_general(x0, x1, (((1,), (1,)), ((), ())),
                                   preferred_element_type=jnp.float32) * scale
        confT = jax.lax.dot_general(x1, x0, (((1,), (1,)), ((), ())),
                                    preferred_element_type=jnp.float32) * scale

        def dual_softmax(cm):
            e2 = jnp.exp(cm - jnp.max(cm, axis=1, keepdims=True))
            sm2 = e2 / jnp.sum(e2, axis=1, keepdims=True)
            e1 = jnp.exp(cm - jnp.max(cm, axis=0, keepdims=True))
            sm1 = e1 / jnp.sum(e1, axis=0, keepdims=True)
            return sm1 * sm2

        iota_s = jax.lax.broadcasted_iota(jnp.int32, (L, L), 1)

        def row_argmax(cm):
            m = jnp.max(cm, axis=1, keepdims=True)
            return jnp.min(jnp.where(cm == m, iota_s, jnp.int32(1 << 30)),
                           axis=1, keepdims=True)

        confm = dual_softmax(conf)
        confmT = dual_softmax(confT)
        idx = row_argmax(confm)
        idy = row_argmax(confmT)

        onehot = (iota_s == idx).astype(jnp.float32)
        gsel = _dot(onehot, x1)
        ft100 = _dot(p_ref[...], (x0 + gsel) * 0.5)

        t32p_ref[...] = (_dot(t32_ref[...], wbt_ref[...]) +
                         _dot(ft100, wbb_ref[...]))
        wal = wal_ref[0:1, :]
        idxf = idx.astype(jnp.float32).astype(jnp.bfloat16).astype(jnp.float32)
        idyf = idy.astype(jnp.float32).astype(jnp.bfloat16).astype(jnp.float32)
        f0_ref[...] = _dot(x0, wat_ref[...]) + idxf * wal
        f1_ref[...] = _dot(x1, wat_ref[...]) + idyf * wal
        tmp_ref[...] = jax.lax.dot_general(
            conf, b_ref[...], (((1,), (0,)), ((), ())),
            preferred_element_type=jnp.float32)

    conf_f_ref[...] = jax.lax.dot_general(
        a_ref[...], tmp_ref[...], (((1,), (0,)), ((), ())),
        preferred_element_type=jnp.float32)


def kernel(feats16_0, feats16_1, feats8_0, feats8_1, feats4_0, feats4_1,
           t64, t32, t16):
    del feats8_0, feats8_1, feats4_0, feats4_1, t64, t16
    a, b, wat, wal, wbt, wbb, p = (jnp.asarray(c) for c in _consts())
    x0 = feats16_0[0]
    x1 = feats16_1[0]
    t32pad = jnp.pad(t32[0], ((0, LTP - LT), (0, 0)))

    grid = (UP // TILE,)
    conf_f, f0, f1, t32p = pl.pallas_call(
        _fused_kernel,
        grid=grid,
        in_specs=[
            pl.BlockSpec((L, C), lambda g: (0, 0)),
            pl.BlockSpec((L, C), lambda g: (0, 0)),
            pl.BlockSpec((LTP, C), lambda g: (0, 0)),
            pl.BlockSpec((TILE, L), lambda g: (g, 0)),
            pl.BlockSpec((L, UP), lambda g: (0, 0)),
            pl.BlockSpec((LTP, L), lambda g: (0, 0)),
            pl.BlockSpec((C, C), lambda g: (0, 0)),
            pl.BlockSpec((8, C), lambda g: (0, 0)),
            pl.BlockSpec((C, C), lambda g: (0, 0)),
            pl.BlockSpec((C, C), lambda g: (0, 0)),
        ],
        out_specs=[
            pl.BlockSpec((TILE, UP), lambda g: (g, 0)),
            pl.BlockSpec((L, C), lambda g: (0, 0)),
            pl.BlockSpec((L, C), lambda g: (0, 0)),
            pl.BlockSpec((LTP, C), lambda g: (0, 0)),
        ],
        out_shape=[
            jax.ShapeDtypeStruct((UP, UP), jnp.float32),
            jax.ShapeDtypeStruct((L, C), jnp.float32),
            jax.ShapeDtypeStruct((L, C), jnp.float32),
            jax.ShapeDtypeStruct((LTP, C), jnp.float32),
        ],
        scratch_shapes=[pltpu.VMEM((L, UP), jnp.float32)],
    )(x0, x1, t32pad, a, b, p, wat, wal, wbt, wbb)

    return (conf_f[None, None], f0[None], f1[None],
            t32p[:LT, None, :])


# speedup vs baseline: 8.5891x; 1.2524x over previous
import functools

import numpy as np
import jax
import jax.numpy as jnp
from jax.experimental import pallas as pl
from jax.experimental.pallas import tpu as pltpu

L = 400
C = 256
UP = 1600
TILE = 160
BAND = 56
LT = 100
LTP = 128
NT = UP // TILE


def _pool1d_matrix(l_in, l_out):
    w = np.zeros((l_in, l_out), dtype=np.float32)
    for i in range(l_out):
        s = (i * l_in) // l_out
        e = -(((-(i + 1)) * l_in) // l_out)
        w[s:e, i] = 1.0 / (e - s)
    return w


def _band_offset(t):
    return min((((TILE * t * (L - 1)) // (UP - 1)) // 8) * 8, L - BAND)


def _bilinear_bands():
    ys = np.linspace(0.0, L - 1.0, UP, dtype=np.float32)
    y0 = np.floor(ys).astype(np.int64)
    y1 = np.minimum(y0 + 1, L - 1)
    wy = ys - y0.astype(np.float32)
    a = np.zeros((UP, L), dtype=np.float32)
    for i in range(UP):
        a[i, y0[i]] += 1.0 - wy[i]
        a[i, y1[i]] += wy[i]
    s = np.zeros((UP, BAND), dtype=np.float32)
    bp = np.zeros((NT * BAND, TILE), dtype=np.float32)
    for t in range(NT):
        off = _band_offset(t)
        s[TILE * t:TILE * (t + 1), :] = a[TILE * t:TILE * (t + 1),
                                          off:off + BAND]
        bp[BAND * t:BAND * (t + 1), :] = a[TILE * t:TILE * (t + 1),
                                           off:off + BAND].T
    return s, bp


def _spatial_pool_matrix():
    p = np.zeros((LTP, L), dtype=np.float32)
    for j in range(LT):
        rr, cc = divmod(j, 10)
        for dr in range(2):
            for dc in range(2):
                p[j, (2 * rr + dr) * 20 + (2 * cc + dc)] = 0.25
    return p


@functools.cache
def _consts():
    s, bp = _bilinear_bands()
    wa = _pool1d_matrix(C + 1, C)
    wb = _pool1d_matrix(2 * C, C)
    wal = np.zeros((8, C), dtype=np.float32)
    wal[0] = wa[C]
    return (s, bp, wa[:C].copy(), wal, wb[:C].copy(), wb[C:].copy(),
            _spatial_pool_matrix())


def _dot(x, y):
    return jax.lax.dot_general(x, y, (((1,), (0,)), ((), ())),
                               preferred_element_type=jnp.float32)


def _fused_kernel(x0_ref, x1_ref, t32_ref, s_ref, bp_ref, p_ref, wat_ref,
                  wal_ref, wbt_ref, wbb_ref,
                  conf_f_ref, f0_ref, f1_ref, t32p_ref, tmp_ref):
    g = pl.program_id(0)

    @pl.when(g == 0)
    def _phase0():
        x0 = x0_ref[...]
        x1 = x1_ref[...]
        scale = 1.0 / (C ** 0.5)
        conf = jax.lax.dot_general(x0, x1, (((1,), (1,)), ((), ())),
                                   preferred_element_type=jnp.float32) * scale
        confT = jax.lax.dot_general(x1, x0, (((1,), (1,)), ((), ())),
                                    preferred_element_type=jnp.float32) * scale

        def dual_softmax(cm):
            e2 = jnp.exp(cm - jnp.max(cm, axis=1, keepdims=True))
            sm2 = e2 / jnp.sum(e2, axis=1, keepdims=True)
            e1 = jnp.exp(cm - jnp.max(cm, axis=0, keepdims=True))
            sm1 = e1 / jnp.sum(e1, axis=0, keepdims=True)
            return sm1 * sm2

        iota_s = jax.lax.broadcasted_iota(jnp.int32, (L, L), 1)

        def row_argmax(cm):
            m = jnp.max(cm, axis=1, keepdims=True)
            return jnp.min(jnp.where(cm == m, iota_s, jnp.int32(1 << 30)),
                           axis=1, keepdims=True)

        confm = dual_softmax(conf)
        confmT = dual_softmax(confT)
        idx = row_argmax(confm)
        idy = row_argmax(confmT)

        onehot = (iota_s == idx).astype(jnp.float32)
        gsel = _dot(onehot, x1)
        ft100 = _dot(p_ref[...], (x0 + gsel) * 0.5)

        t32p_ref[...] = (_dot(t32_ref[...], wbt_ref[...]) +
                         _dot(ft100, wbb_ref[...]))
        wal = wal_ref[0:1, :]
        idxf = idx.astype(jnp.float32).astype(jnp.bfloat16).astype(jnp.float32)
        idyf = idy.astype(jnp.float32).astype(jnp.bfloat16).astype(jnp.float32)
        f0_ref[...] = _dot(x0, wat_ref[...]) + idxf * wal
        f1_ref[...] = _dot(x1, wat_ref[...]) + idyf * wal

        for t in range(NT):
            off = _band_offset(t)
            tmp_ref[:, TILE * t:TILE * (t + 1)] = _dot(
                conf[:, off:off + BAND],
                bp_ref[BAND * t:BAND * (t + 1), :])

    start = jnp.minimum(((TILE * g * (L - 1)) // (UP - 1)) // 8 * 8, L - BAND)
    conf_f_ref[...] = _dot(s_ref[...], tmp_ref[pl.ds(start, BAND), :])


def kernel(feats16_0, feats16_1, feats8_0, feats8_1, feats4_0, feats4_1,
           t64, t32, t16):
    del feats8_0, feats8_1, feats4_0, feats4_1, t64, t16
    s, bp, wat, wal, wbt, wbb, p = (jnp.asarray(c) for c in _consts())
    x0 = feats16_0[0]
    x1 = feats16_1[0]
    t32pad = jnp.pad(t32[0], ((0, LTP - LT), (0, 0)))

    conf_f, f0, f1, t32p = pl.pallas_call(
        _fused_kernel,
        grid=(NT,),
        in_specs=[
            pl.BlockSpec((L, C), lambda g: (0, 0)),
            pl.BlockSpec((L, C), lambda g: (0, 0)),
            pl.BlockSpec((LTP, C), lambda g: (0, 0)),
            pl.BlockSpec((TILE, BAND), lambda g: (g, 0)),
            pl.BlockSpec((NT * BAND, TILE), lambda g: (0, 0)),
            pl.BlockSpec((LTP, L), lambda g: (0, 0)),
            pl.BlockSpec((C, C), lambda g: (0, 0)),
            pl.BlockSpec((8, C), lambda g: (0, 0)),
            pl.BlockSpec((C, C), lambda g: (0, 0)),
            pl.BlockSpec((C, C), lambda g: (0, 0)),
        ],
        out_specs=[
            pl.BlockSpec((TILE, UP), lambda g: (g, 0)),
            pl.BlockSpec((L, C), lambda g: (0, 0)),
            pl.BlockSpec((L, C), lambda g: (0, 0)),
            pl.BlockSpec((LTP, C), lambda g: (0, 0)),
        ],
        out_shape=[
            jax.ShapeDtypeStruct((UP, UP), jnp.float32),
            jax.ShapeDtypeStruct((L, C), jnp.float32),
            jax.ShapeDtypeStruct((L, C), jnp.float32),
            jax.ShapeDtypeStruct((LTP, C), jnp.float32),
        ],
        scratch_shapes=[pltpu.VMEM((L, UP), jnp.float32)],
    )(x0, x1, t32pad, s, bp, p, wat, wal, wbt, wbb)

    return (conf_f[None, None], f0[None], f1[None],
            t32p[:LT, None, :])
